# baseline (speedup 1.0000x reference)
import functools

import jax
import jax.numpy as jnp
from jax import lax
from jax.experimental import pallas as pl
from jax.experimental.pallas import tpu as pltpu
from jax.experimental.pallas import tpu_sc as plsc

NUM_ELEMENTS = 118
PAD_ROWS = 128
EMBED_DIM = 64

_NC = 2
_NS = 16
_NW = _NC * _NS
_L = 16


def _make_embed(N: int, M: int, R: int):
    B = N * M
    C = R * M
    assert B % (_NW * C) == 0 and C % _L == 0
    rows_per_w = N // _NW
    per_w = B // _NW
    n_chunks = per_w // C
    assert n_chunks % 2 == 0
    n_groups = C // _L
    mesh = plsc.VectorSubcoreMesh(core_axis_name="c", subcore_axis_name="s")

    @functools.partial(
        pl.kernel,
        mesh=mesh,
        out_type=jax.ShapeDtypeStruct((N, M, EMBED_DIM), jnp.float32),
        scratch_types=[
            pltpu.VMEM((PAD_ROWS * EMBED_DIM,), jnp.float32),
            pltpu.VMEM((C,), jnp.int32),
            pltpu.VMEM((C,), jnp.int32),
            pltpu.VMEM((C, EMBED_DIM), jnp.float32),
            pltpu.VMEM((C, EMBED_DIM), jnp.float32),
            pltpu.SemaphoreType.DMA,
            pltpu.SemaphoreType.DMA,
            pltpu.SemaphoreType.DMA,
            pltpu.SemaphoreType.DMA,
            pltpu.SemaphoreType.DMA,
        ],
    )
    def emb(table_hbm, idx_hbm, out_hbm, table_v, idx0, idx1, rows0, rows1,
            isem0, isem1, tsem, osem0, osem1):
        wid = lax.axis_index("s") * _NC + lax.axis_index("c")
        base = wid * per_w
        row0 = wid * rows_per_w
        idx_bufs = (idx0, idx1)
        rows_bufs = (rows0, rows1)
        isems = (isem0, isem1)
        osems = (osem0, osem1)

        pltpu.async_copy(table_hbm, table_v, tsem)
        idx_start_0 = pltpu.async_copy(
            idx_hbm.at[pl.ds(base, C)], idx_bufs[0], isems[0])
        del idx_start_0
        pltpu.async_copy(idx_hbm.at[pl.ds(base + C, C)], idx_bufs[1], isems[1])
        pltpu.make_async_copy(table_hbm, table_v, tsem).wait()

        def idx_start(g, b):
            pltpu.async_copy(idx_hbm.at[pl.ds(base + g * C, C)], idx_bufs[b], isems[b])

        def idx_wait(b):
            pltpu.make_async_copy(idx_hbm.at[pl.ds(0, C)], idx_bufs[b], isems[b]).wait()

        def out_start(g, b):
            r = row0 + g * R
            for j in range(R):
                pltpu.async_copy(rows_bufs[b].at[pl.ds(j * M, M)],
                                 out_hbm.at[r + j], osems[b])

        def out_wait(b):
            for j in range(R):
                pltpu.make_async_copy(rows_bufs[b].at[pl.ds(j * M, M)],
                                      out_hbm.at[0], osems[b]).wait()

        def expand(b):
            rows_b = rows_bufs[b]
            idx_b = idx_bufs[b]

            @plsc.parallel_loop(0, n_groups, 1, unroll=2)
            def group(q):
                zv = idx_b[pl.ds(q * _L, _L)] * EMBED_DIM
                srcs = [zv[k] for k in range(_L)]
                for k in range(_L):
                    p = q * _L + k
                    for m in range(0, EMBED_DIM, _L):
                        rows_b[p, pl.ds(m, _L)] = table_v[pl.ds(srcs[k] + m, _L)]

        def body(i, carry):
            for b in (0, 1):
                g = 2 * i + b
                idx_wait(b)
                @pl.when(g >= 2)
                def _():
                    out_wait(b)
                expand(b)
                @pl.when(g + 2 < n_chunks)
                def _():
                    idx_start(g + 2, b)
                out_start(g, b)
            return carry

        lax.fori_loop(0, n_chunks // 2, body, 0)
        out_wait(0)
        out_wait(1)

    return emb


def kernel(z, table):
    n, m = z.shape
    zf = z.reshape(n * m).astype(jnp.int32)
    tp = jnp.pad(table, ((0, PAD_ROWS - NUM_ELEMENTS), (0, 0))).reshape(-1)
    return _make_embed(n, m, 2)(tp, zf)


# speedup vs baseline: 1.0023x; 1.0023x over previous
import functools

import jax
import jax.numpy as jnp
from jax import lax
from jax.experimental import pallas as pl
from jax.experimental.pallas import tpu as pltpu
from jax.experimental.pallas import tpu_sc as plsc

NUM_ELEMENTS = 118
PAD_ROWS = 128
EMBED_DIM = 64

_NC = 2
_NS = 16
_NW = _NC * _NS
_L = 16


def _make_embed(N: int, M: int, R: int):
    B = N * M
    C = R * M
    assert B % (_NW * C) == 0 and C % _L == 0
    rows_per_w = N // _NW
    per_w = B // _NW
    n_chunks = per_w // C
    assert n_chunks % 2 == 0
    n_groups = C // _L
    mesh = plsc.VectorSubcoreMesh(core_axis_name="c", subcore_axis_name="s")

    @functools.partial(
        pl.kernel,
        mesh=mesh,
        out_type=jax.ShapeDtypeStruct((N, M, EMBED_DIM), jnp.float32),
        compiler_params=pltpu.CompilerParams(use_tc_tiling_on_sc=True),
        scratch_types=[
            pltpu.VMEM((PAD_ROWS * EMBED_DIM,), jnp.float32),
            pltpu.VMEM((C,), jnp.int32),
            pltpu.VMEM((C,), jnp.int32),
            pltpu.VMEM((C, EMBED_DIM), jnp.float32),
            pltpu.VMEM((C, EMBED_DIM), jnp.float32),
            pltpu.SemaphoreType.DMA,
            pltpu.SemaphoreType.DMA,
            pltpu.SemaphoreType.DMA,
            pltpu.SemaphoreType.DMA,
            pltpu.SemaphoreType.DMA,
        ],
    )
    def emb(table_hbm, idx_hbm, out_hbm, table_v, idx0, idx1, rows0, rows1,
            isem0, isem1, tsem, osem0, osem1):
        wid = lax.axis_index("s") * _NC + lax.axis_index("c")
        base = wid * per_w
        row0 = wid * rows_per_w
        idx_bufs = (idx0, idx1)
        rows_bufs = (rows0, rows1)
        isems = (isem0, isem1)
        osems = (osem0, osem1)

        pltpu.async_copy(table_hbm, table_v, tsem)
        idx_start_0 = pltpu.async_copy(
            idx_hbm.at[pl.ds(base, C)], idx_bufs[0], isems[0])
        del idx_start_0
        pltpu.async_copy(idx_hbm.at[pl.ds(base + C, C)], idx_bufs[1], isems[1])
        pltpu.make_async_copy(table_hbm, table_v, tsem).wait()

        def idx_start(g, b):
            pltpu.async_copy(idx_hbm.at[pl.ds(base + g * C, C)], idx_bufs[b], isems[b])

        def idx_wait(b):
            pltpu.make_async_copy(idx_hbm.at[pl.ds(0, C)], idx_bufs[b], isems[b]).wait()

        def out_start(g, b):
            r = row0 + g * R
            for j in range(R):
                pltpu.async_copy(rows_bufs[b].at[pl.ds(j * M, M)],
                                 out_hbm.at[r + j], osems[b])

        def out_wait(b):
            for j in range(R):
                pltpu.make_async_copy(rows_bufs[b].at[pl.ds(j * M, M)],
                                      out_hbm.at[0], osems[b]).wait()

        def expand(b):
            rows_b = rows_bufs[b]
            idx_b = idx_bufs[b]

            @plsc.parallel_loop(0, n_groups, 1, unroll=2)
            def group(q):
                zv = idx_b[pl.ds(q * _L, _L)] * EMBED_DIM
                srcs = [zv[k] for k in range(_L)]
                for k in range(_L):
                    p = q * _L + k
                    for m in range(0, EMBED_DIM, _L):
                        rows_b[p, pl.ds(m, _L)] = table_v[pl.ds(srcs[k] + m, _L)]

        def body(i, carry):
            for b in (0, 1):
                g = 2 * i + b
                idx_wait(b)
                @pl.when(g >= 2)
                def _():
                    out_wait(b)
                expand(b)
                @pl.when(g + 2 < n_chunks)
                def _():
                    idx_start(g + 2, b)
                out_start(g, b)
            return carry

        lax.fori_loop(0, n_chunks // 2, body, 0)
        out_wait(0)
        out_wait(1)

    return emb


def kernel(z, table):
    n, m = z.shape
    zf = z.reshape(n * m).astype(jnp.int32)
    tp = jnp.pad(table, ((0, PAD_ROWS - NUM_ELEMENTS), (0, 0))).reshape(-1)
    return _make_embed(n, m, 2)(tp, zf)
